# baseline (speedup 1.0000x reference)
import functools

import jax
import jax.numpy as jnp
from jax import lax
from jax.experimental import pallas as pl
from jax.experimental.pallas import tpu as pltpu
from jax.experimental.pallas import tpu_sc as plsc

D = 2048
S = 100
NUM_LEVELS = 100
B = 512
C = 26

LANES = 16
SP = 112
NW = 32
BPW = B // NW
DC = 256
NCHUNK = D // DC
TPC = DC // LANES


def _sc_encode(xt_pad, id_pad, level_weight):
  mesh = plsc.VectorSubcoreMesh(core_axis_name="c", subcore_axis_name="s")

  @functools.partial(
      pl.kernel,
      out_type=jax.ShapeDtypeStruct((B, D), jnp.float32),
      mesh=mesh,
      compiler_params=pltpu.CompilerParams(use_tc_tiling_on_sc=False),
      scratch_types=[
          pltpu.VMEM((SP, BPW), jnp.float32),
          pltpu.VMEM((SP, BPW), jnp.int32),
          pltpu.VMEM((NUM_LEVELS, DC), jnp.float32),
          pltpu.VMEM((SP, DC), jnp.float32),
          pltpu.VMEM((BPW, DC), jnp.float32),
      ],
  )
  def enc_kernel(xt_hbm, id_hbm, lvl_hbm, out_hbm, xt_v, idx_v, lvl_v, id_v,
                 acc_v):
    wid = lax.axis_index("s") * 2 + lax.axis_index("c")
    base = wid * BPW

    pltpu.sync_copy(xt_hbm.at[:, pl.ds(base, BPW)], xt_v)

    def idx_body(s, _):
      v = xt_v[s, :] * jnp.float32(NUM_LEVELS - 1) + jnp.float32(0.5)
      iv = v.astype(jnp.int32)
      iv = jnp.minimum(jnp.maximum(iv, 0), NUM_LEVELS - 1)
      idx_v[s, :] = iv
      return 0

    lax.fori_loop(0, SP, idx_body, 0)

    def chunk_body(ci, _):
      col = ci * DC
      pltpu.sync_copy(lvl_hbm.at[:, pl.ds(col, DC)], lvl_v)
      pltpu.sync_copy(id_hbm.at[:, pl.ds(col, DC)], id_v)

      def zero_body(b, _):
        for t in range(TPC):
          acc_v[b, pl.ds(t * LANES, LANES)] = jnp.zeros((LANES,), jnp.float32)
        return 0

      lax.fori_loop(0, BPW, zero_body, 0)

      def s_body(s, _):
        vidx = idx_v[s, :]
        idrow = [id_v[s, pl.ds(t * LANES, LANES)] for t in range(TPC)]
        for b in range(BPW):
          i = vidx[b]
          for t in range(TPC):
            plsc.addupdate(
                acc_v.at[b, pl.ds(t * LANES, LANES)],
                lvl_v[i, pl.ds(t * LANES, LANES)] * idrow[t],
            )
        return 0

      lax.fori_loop(0, SP, s_body, 0)

      def sign_body(b, _):
        for t in range(TPC):
          a = acc_v[b, pl.ds(t * LANES, LANES)]
          acc_v[b, pl.ds(t * LANES, LANES)] = jnp.where(
              a > 0, jnp.float32(1.0), jnp.float32(-1.0))
        return 0

      lax.fori_loop(0, BPW, sign_body, 0)
      pltpu.sync_copy(acc_v, out_hbm.at[pl.ds(base, BPW), pl.ds(col, DC)])
      return 0

    lax.fori_loop(0, NCHUNK, chunk_body, 0)

  return enc_kernel(xt_pad, id_pad, level_weight)


def _tc_classify(enc, wt_pad):
  cp = wt_pad.shape[1]

  def body(enc_ref, w_ref, out_ref):
    out_ref[:] = jnp.dot(enc_ref[:], w_ref[:],
                         preferred_element_type=jnp.float32)

  return pl.pallas_call(
      body,
      out_shape=jax.ShapeDtypeStruct((B, cp), jnp.float32),
  )(enc, wt_pad)


@jax.jit
def kernel(x, id_weight, level_weight, classify_weight):
  xt_pad = jnp.zeros((SP, B), jnp.float32).at[:S].set(x.T)
  id_pad = jnp.zeros((SP, D), jnp.float32).at[:S].set(id_weight)
  enc = _sc_encode(xt_pad, id_pad, level_weight)
  cp = 128
  wt_pad = jnp.zeros((D, cp), jnp.float32).at[:, :C].set(classify_weight.T)
  logit = _tc_classify(enc, wt_pad)
  return logit[:, :C]


# speedup vs baseline: 3.4779x; 3.4779x over previous
import functools

import jax
import jax.numpy as jnp
from jax import lax
from jax.experimental import pallas as pl
from jax.experimental.pallas import tpu as pltpu
from jax.experimental.pallas import tpu_sc as plsc

D = 2048
S = 100
NUM_LEVELS = 100
B = 512
C = 26

LANES = 16
SP = 112
NW = 32
BPW = B // NW
DC = 256
NCHUNK = D // DC
TPC = DC // LANES


def _sc_encode(xt_pad, id_pad, level_weight):
  mesh = plsc.VectorSubcoreMesh(core_axis_name="c", subcore_axis_name="s")

  @functools.partial(
      pl.kernel,
      out_type=jax.ShapeDtypeStruct((B, D), jnp.float32),
      mesh=mesh,
      compiler_params=pltpu.CompilerParams(use_tc_tiling_on_sc=False),
      scratch_types=[
          pltpu.VMEM((SP, BPW), jnp.float32),
          pltpu.VMEM((SP, BPW), jnp.int32),
          pltpu.VMEM((NUM_LEVELS, DC), jnp.float32),
          pltpu.VMEM((SP, DC), jnp.float32),
          pltpu.VMEM((BPW, DC), jnp.float32),
      ],
  )
  def enc_kernel(xt_hbm, id_hbm, lvl_hbm, out_hbm, xt_v, idx_v, lvl_v, id_v,
                 acc_v):
    wid = lax.axis_index("s") * 2 + lax.axis_index("c")
    base = wid * BPW

    pltpu.sync_copy(xt_hbm.at[:, pl.ds(base, BPW)], xt_v)

    def idx_body(s, _):
      v = xt_v[s, :] * jnp.float32(NUM_LEVELS - 1) + jnp.float32(0.5)
      iv = v.astype(jnp.int32)
      iv = jnp.minimum(jnp.maximum(iv, 0), NUM_LEVELS - 1)
      idx_v[s, :] = iv
      return 0

    lax.fori_loop(0, S, idx_body, 0)

    def chunk_body(ci, _):
      col = ci * DC
      pltpu.sync_copy(lvl_hbm.at[:, pl.ds(col, DC)], lvl_v)
      pltpu.sync_copy(id_hbm.at[:, pl.ds(col, DC)], id_v)

      def zero_body(b, _):
        for t in range(TPC):
          acc_v[b, pl.ds(t * LANES, LANES)] = jnp.zeros((LANES,), jnp.float32)
        return 0

      lax.fori_loop(0, BPW, zero_body, 0)

      @plsc.parallel_loop(0, S, unroll=2)
      def _(s):
        vidx = idx_v[s, :]
        ib = [vidx[b] for b in range(BPW)]
        idrow = [id_v[s, pl.ds(t * LANES, LANES)] for t in range(TPC)]

        def load_row(b):
          return [lvl_v[ib[b], pl.ds(t * LANES, LANES)] for t in range(TPC)]

        lrow = load_row(0)
        for b in range(BPW):
          prods = [lrow[t] * idrow[t] for t in range(TPC)]
          nxt = []
          for t in range(TPC):
            if b + 1 < BPW:
              nxt.append(lvl_v[ib[b + 1], pl.ds(t * LANES, LANES)])
            plsc.addupdate(acc_v.at[b, pl.ds(t * LANES, LANES)], prods[t])
          lrow = nxt

      def sign_body(b, _):
        for t in range(TPC):
          a = acc_v[b, pl.ds(t * LANES, LANES)]
          acc_v[b, pl.ds(t * LANES, LANES)] = jnp.where(
              a > 0, jnp.float32(1.0), jnp.float32(-1.0))
        return 0

      lax.fori_loop(0, BPW, sign_body, 0)
      pltpu.sync_copy(acc_v, out_hbm.at[pl.ds(base, BPW), pl.ds(col, DC)])
      return 0

    lax.fori_loop(0, NCHUNK, chunk_body, 0)

  return enc_kernel(xt_pad, id_pad, level_weight)


def _tc_classify(enc, wt_pad):
  cp = wt_pad.shape[1]

  def body(enc_ref, w_ref, out_ref):
    out_ref[:] = jnp.dot(enc_ref[:], w_ref[:],
                         preferred_element_type=jnp.float32)

  return pl.pallas_call(
      body,
      out_shape=jax.ShapeDtypeStruct((B, cp), jnp.float32),
  )(enc, wt_pad)


@jax.jit
def kernel(x, id_weight, level_weight, classify_weight):
  xt_pad = jnp.zeros((SP, B), jnp.float32).at[:S].set(x.T)
  id_pad = jnp.zeros((SP, D), jnp.float32).at[:S].set(id_weight)
  enc = _sc_encode(xt_pad, id_pad, level_weight)
  cp = 128
  wt_pad = jnp.zeros((D, cp), jnp.float32).at[:, :C].set(classify_weight.T)
  logit = _tc_classify(enc, wt_pad)
  return logit[:, :C]


# speedup vs baseline: 6.9883x; 2.0093x over previous
import functools

import jax
import jax.numpy as jnp
from jax import lax
from jax.experimental import pallas as pl
from jax.experimental.pallas import tpu as pltpu
from jax.experimental.pallas import tpu_sc as plsc

D = 2048
S = 100
NUM_LEVELS = 100
B = 512
C = 26

LANES = 16
BLANES = 32
SP = 112
NW = 32
BPW = B // NW
DC = 128
NCHUNK = D // DC
TPB = DC // BLANES
G = 5


def _sc_encode(xt_pad, id_bf, lvl_bf):
  mesh = plsc.VectorSubcoreMesh(core_axis_name="c", subcore_axis_name="s")

  @functools.partial(
      pl.kernel,
      out_type=jax.ShapeDtypeStruct((B, D), jnp.bfloat16),
      mesh=mesh,
      compiler_params=pltpu.CompilerParams(use_tc_tiling_on_sc=False),
      scratch_types=[
          pltpu.VMEM((SP, BPW), jnp.float32),
          pltpu.VMEM((S, BPW), jnp.int32),
          pltpu.VMEM((NUM_LEVELS, DC), jnp.bfloat16),
          pltpu.VMEM((S, DC), jnp.bfloat16),
          pltpu.VMEM((BPW, DC), jnp.bfloat16),
      ],
  )
  def enc_kernel(xt_hbm, id_hbm, lvl_hbm, out_hbm, xt_v, idx_v, lvl_v, id_v,
                 acc_v):
    wid = lax.axis_index("s") * 2 + lax.axis_index("c")
    base = wid * BPW

    pltpu.sync_copy(xt_hbm.at[:, pl.ds(base, BPW)], xt_v)

    def idx_body(s, _):
      v = xt_v[s, :] * jnp.float32(NUM_LEVELS - 1) + jnp.float32(0.5)
      iv = v.astype(jnp.int32)
      iv = jnp.minimum(jnp.maximum(iv, 0), NUM_LEVELS - 1)
      idx_v[s, :] = iv
      return 0

    lax.fori_loop(0, S, idx_body, 0)

    def chunk_body(ci, _):
      col = ci * DC
      pltpu.sync_copy(lvl_hbm.at[:, pl.ds(col, DC)], lvl_v)
      pltpu.sync_copy(id_hbm.at[:, pl.ds(col, DC)], id_v)

      zero = jnp.zeros((BLANES,), jnp.bfloat16)

      def zero_body(b, _):
        for t in range(TPB):
          acc_v[b, pl.ds(t * BLANES, BLANES)] = zero
        return 0

      lax.fori_loop(0, BPW, zero_body, 0)

      def g_body(gi, _):
        s0 = gi * G
        vidx = [idx_v[s0 + g, :] for g in range(G)]
        idrow = [
            [id_v[s0 + g, pl.ds(t * BLANES, BLANES)] for t in range(TPB)]
            for g in range(G)
        ]
        for b in range(BPW):
          ib = [vidx[g][b] for g in range(G)]
          accs = [acc_v[b, pl.ds(t * BLANES, BLANES)] for t in range(TPB)]
          for g in range(G):
            lr = [lvl_v[ib[g], pl.ds(t * BLANES, BLANES)] for t in range(TPB)]
            for t in range(TPB):
              accs[t] = accs[t] + lr[t] * idrow[g][t]
          for t in range(TPB):
            acc_v[b, pl.ds(t * BLANES, BLANES)] = accs[t]
        return 0

      lax.fori_loop(0, S // G, g_body, 0)

      one = jnp.float32(1.0).astype(jnp.bfloat16)
      mone = jnp.float32(-1.0).astype(jnp.bfloat16)
      zbf = jnp.float32(0.0).astype(jnp.bfloat16)

      def sign_body(b, _):
        for t in range(TPB):
          a = acc_v[b, pl.ds(t * BLANES, BLANES)]
          acc_v[b, pl.ds(t * BLANES, BLANES)] = jnp.where(a > zbf, one, mone)
        return 0

      lax.fori_loop(0, BPW, sign_body, 0)
      pltpu.sync_copy(acc_v, out_hbm.at[pl.ds(base, BPW), pl.ds(col, DC)])
      return 0

    lax.fori_loop(0, NCHUNK, chunk_body, 0)

  return enc_kernel(xt_pad, id_bf, lvl_bf)


def _tc_classify(enc, wt_pad):
  cp = wt_pad.shape[1]

  def body(enc_ref, w_ref, out_ref):
    out_ref[:] = jnp.dot(enc_ref[:].astype(jnp.float32), w_ref[:],
                         preferred_element_type=jnp.float32)

  return pl.pallas_call(
      body,
      out_shape=jax.ShapeDtypeStruct((B, cp), jnp.float32),
  )(enc, wt_pad)


@jax.jit
def kernel(x, id_weight, level_weight, classify_weight):
  xt_pad = jnp.zeros((SP, B), jnp.float32).at[:S].set(x.T)
  id_bf = id_weight.astype(jnp.bfloat16)
  lvl_bf = level_weight.astype(jnp.bfloat16)
  enc = _sc_encode(xt_pad, id_bf, lvl_bf)
  cp = 128
  wt_pad = jnp.zeros((D, cp), jnp.float32).at[:, :C].set(classify_weight.T)
  logit = _tc_classify(enc, wt_pad)
  return logit[:, :C]


# speedup vs baseline: 7.3564x; 1.0527x over previous
import functools

import jax
import jax.numpy as jnp
from jax import lax
from jax.experimental import pallas as pl
from jax.experimental.pallas import tpu as pltpu
from jax.experimental.pallas import tpu_sc as plsc

D = 2048
S = 100
NUM_LEVELS = 100
B = 512
C = 26

LANES = 16
BLANES = 32
SP = 112
NW = 32
BPW = B // NW
DC = 128
NCHUNK = D // DC
TPB = DC // BLANES
G = 8


def _sc_encode(xt_pad, id_bf, lvl_bf):
  mesh = plsc.VectorSubcoreMesh(core_axis_name="c", subcore_axis_name="s")

  @functools.partial(
      pl.kernel,
      out_type=jax.ShapeDtypeStruct((B, D), jnp.bfloat16),
      mesh=mesh,
      compiler_params=pltpu.CompilerParams(use_tc_tiling_on_sc=False),
      scratch_types=[
          pltpu.VMEM((SP, BPW), jnp.float32),
          pltpu.VMEM((S, BPW), jnp.int32),
          pltpu.VMEM((NUM_LEVELS, DC), jnp.bfloat16),
          pltpu.VMEM((S, DC), jnp.bfloat16),
          pltpu.VMEM((BPW, DC), jnp.bfloat16),
      ],
  )
  def enc_kernel(xt_hbm, id_hbm, lvl_hbm, out_hbm, xt_v, idx_v, lvl_v, id_v,
                 acc_v):
    wid = lax.axis_index("s") * 2 + lax.axis_index("c")
    base = wid * BPW

    pltpu.sync_copy(xt_hbm.at[:, pl.ds(base, BPW)], xt_v)

    def idx_body(s, _):
      v = xt_v[s, :] * jnp.float32(NUM_LEVELS - 1) + jnp.float32(0.5)
      iv = v.astype(jnp.int32)
      iv = jnp.minimum(jnp.maximum(iv, 0), NUM_LEVELS - 1)
      idx_v[s, :] = iv
      return 0

    lax.fori_loop(0, S, idx_body, 0)

    def chunk_body(ci, _):
      col = ci * DC
      pltpu.sync_copy(lvl_hbm.at[:, pl.ds(col, DC)], lvl_v)
      pltpu.sync_copy(id_hbm.at[:, pl.ds(col, DC)], id_v)

      zero = jnp.zeros((BLANES,), jnp.bfloat16)

      def zero_body(b, _):
        for t in range(TPB):
          acc_v[b, pl.ds(t * BLANES, BLANES)] = zero
        return 0

      lax.fori_loop(0, BPW, zero_body, 0)

      def g_core(s0, glen):
        vidx = [idx_v[s0 + g, :] for g in range(glen)]
        idrow = [
            [id_v[s0 + g, pl.ds(t * BLANES, BLANES)] for t in range(TPB)]
            for g in range(glen)
        ]
        for b in range(BPW):
          ib = [vidx[g][b] for g in range(glen)]
          accs = [acc_v[b, pl.ds(t * BLANES, BLANES)] for t in range(TPB)]
          for g in range(glen):
            lr = [lvl_v[ib[g], pl.ds(t * BLANES, BLANES)] for t in range(TPB)]
            for t in range(TPB):
              accs[t] = accs[t] + lr[t] * idrow[g][t]
          for t in range(TPB):
            acc_v[b, pl.ds(t * BLANES, BLANES)] = accs[t]

      def g_body(gi, _):
        g_core(gi * G, G)
        return 0

      lax.fori_loop(0, (S // G), g_body, 0)
      if S % G:
        g_core(S - S % G, S % G)

      one = jnp.float32(1.0).astype(jnp.bfloat16)
      mone = jnp.float32(-1.0).astype(jnp.bfloat16)
      zbf = jnp.float32(0.0).astype(jnp.bfloat16)

      def sign_body(b, _):
        for t in range(TPB):
          a = acc_v[b, pl.ds(t * BLANES, BLANES)]
          acc_v[b, pl.ds(t * BLANES, BLANES)] = jnp.where(a > zbf, one, mone)
        return 0

      lax.fori_loop(0, BPW, sign_body, 0)
      pltpu.sync_copy(acc_v, out_hbm.at[pl.ds(base, BPW), pl.ds(col, DC)])
      return 0

    lax.fori_loop(0, NCHUNK, chunk_body, 0)

  return enc_kernel(xt_pad, id_bf, lvl_bf)


def _tc_classify(enc, wt_pad):
  cp = wt_pad.shape[1]

  def body(enc_ref, w_ref, out_ref):
    out_ref[:] = jnp.dot(enc_ref[:].astype(jnp.float32), w_ref[:],
                         preferred_element_type=jnp.float32)

  return pl.pallas_call(
      body,
      out_shape=jax.ShapeDtypeStruct((B, cp), jnp.float32),
  )(enc, wt_pad)


@jax.jit
def kernel(x, id_weight, level_weight, classify_weight):
  xt_pad = jnp.zeros((SP, B), jnp.float32).at[:S].set(x.T)
  id_bf = id_weight.astype(jnp.bfloat16)
  lvl_bf = level_weight.astype(jnp.bfloat16)
  enc = _sc_encode(xt_pad, id_bf, lvl_bf)
  cp = 128
  wt_pad = jnp.zeros((D, cp), jnp.float32).at[:, :C].set(classify_weight.T)
  logit = _tc_classify(enc, wt_pad)
  return logit[:, :C]


# speedup vs baseline: 10.1872x; 1.3848x over previous
import functools

import jax
import jax.numpy as jnp
from jax import lax
from jax.experimental import pallas as pl
from jax.experimental.pallas import tpu as pltpu
from jax.experimental.pallas import tpu_sc as plsc

D = 2048
S = 100
NUM_LEVELS = 100
B = 512
C = 26

B_SC = 256
B_TC = B - B_SC

LANES = 16
BLANES = 32
SP = 112
NW = 32
BPW = B_SC // NW
DC = 128
NCHUNK = D // DC
TPB = DC // BLANES
G = 8

LP = 128
DT = 512


def _sc_encode(xt_pad, id_bf, lvl_bf):
  mesh = plsc.VectorSubcoreMesh(core_axis_name="c", subcore_axis_name="s")

  @functools.partial(
      pl.kernel,
      out_type=jax.ShapeDtypeStruct((B_SC, D), jnp.bfloat16),
      mesh=mesh,
      compiler_params=pltpu.CompilerParams(use_tc_tiling_on_sc=False),
      scratch_types=[
          pltpu.VMEM((SP, BPW), jnp.float32),
          pltpu.VMEM((S, BPW), jnp.int32),
          pltpu.VMEM((NUM_LEVELS, DC), jnp.bfloat16),
          pltpu.VMEM((S, DC), jnp.bfloat16),
          pltpu.VMEM((BPW, DC), jnp.bfloat16),
      ],
  )
  def enc_kernel(xt_hbm, id_hbm, lvl_hbm, out_hbm, xt_v, idx_v, lvl_v, id_v,
                 acc_v):
    wid = lax.axis_index("s") * 2 + lax.axis_index("c")
    base = wid * BPW

    pltpu.sync_copy(xt_hbm.at[:, pl.ds(base, BPW)], xt_v)

    def idx_body(s, _):
      v = xt_v[s, :] * jnp.float32(NUM_LEVELS - 1) + jnp.float32(0.5)
      iv = v.astype(jnp.int32)
      iv = jnp.minimum(jnp.maximum(iv, 0), NUM_LEVELS - 1)
      idx_v[s, :] = iv
      return 0

    lax.fori_loop(0, S, idx_body, 0)

    def chunk_body(ci, _):
      col = ci * DC
      pltpu.sync_copy(lvl_hbm.at[:, pl.ds(col, DC)], lvl_v)
      pltpu.sync_copy(id_hbm.at[:, pl.ds(col, DC)], id_v)

      zero = jnp.zeros((BLANES,), jnp.bfloat16)

      def zero_body(b, _):
        for t in range(TPB):
          acc_v[b, pl.ds(t * BLANES, BLANES)] = zero
        return 0

      lax.fori_loop(0, BPW, zero_body, 0)

      def g_core(s0, glen):
        vidx = [idx_v[s0 + g, :] for g in range(glen)]
        idrow = [
            [id_v[s0 + g, pl.ds(t * BLANES, BLANES)] for t in range(TPB)]
            for g in range(glen)
        ]
        for b in range(BPW):
          ib = [vidx[g][b] for g in range(glen)]
          accs = [acc_v[b, pl.ds(t * BLANES, BLANES)] for t in range(TPB)]
          for g in range(glen):
            lr = [lvl_v[ib[g], pl.ds(t * BLANES, BLANES)] for t in range(TPB)]
            for t in range(TPB):
              accs[t] = accs[t] + lr[t] * idrow[g][t]
          for t in range(TPB):
            acc_v[b, pl.ds(t * BLANES, BLANES)] = accs[t]

      def g_body(gi, _):
        g_core(gi * G, G)
        return 0

      lax.fori_loop(0, (S // G), g_body, 0)
      if S % G:
        g_core(S - S % G, S % G)

      one = jnp.float32(1.0).astype(jnp.bfloat16)
      mone = jnp.float32(-1.0).astype(jnp.bfloat16)
      zbf = jnp.float32(0.0).astype(jnp.bfloat16)

      def sign_body(b, _):
        for t in range(TPB):
          a = acc_v[b, pl.ds(t * BLANES, BLANES)]
          acc_v[b, pl.ds(t * BLANES, BLANES)] = jnp.where(a > zbf, one, mone)
        return 0

      lax.fori_loop(0, BPW, sign_body, 0)
      pltpu.sync_copy(acc_v, out_hbm.at[pl.ds(base, BPW), pl.ds(col, DC)])
      return 0

    lax.fori_loop(0, NCHUNK, chunk_body, 0)

  return enc_kernel(xt_pad, id_bf, lvl_bf)


def _tc_encode(x_tc, id_bf, lvl_pad):
  def body(x_ref, id_ref, lvl_ref, out_ref):
    idxf = x_ref[:] * jnp.float32(NUM_LEVELS - 1) + jnp.float32(0.5)
    idxi = jnp.clip(idxf.astype(jnp.int32), 0, NUM_LEVELS - 1)
    lcol = lax.broadcasted_iota(jnp.int32, (B_TC, LP), 1)
    lvl = lvl_ref[:]
    acc = jnp.zeros((B_TC, DT), jnp.float32)
    for s in range(S):
      oh = (idxi[:, s:s + 1] == lcol).astype(jnp.bfloat16)
      ts = lvl * id_ref[s:s + 1, :].astype(jnp.bfloat16)
      acc = acc + jnp.dot(oh, ts, preferred_element_type=jnp.float32)
    out_ref[:] = jnp.where(acc > 0, jnp.float32(1.0),
                           jnp.float32(-1.0)).astype(jnp.bfloat16)

  return pl.pallas_call(
      body,
      grid=(D // DT,),
      in_specs=[
          pl.BlockSpec((B_TC, S), lambda i: (0, 0)),
          pl.BlockSpec((S, DT), lambda i: (0, i)),
          pl.BlockSpec((LP, DT), lambda i: (0, i)),
      ],
      out_specs=pl.BlockSpec((B_TC, DT), lambda i: (0, i)),
      out_shape=jax.ShapeDtypeStruct((B_TC, D), jnp.bfloat16),
  )(x_tc, id_bf, lvl_pad)


def _tc_classify(enc, wt_pad):
  cp = wt_pad.shape[1]

  def body(enc_ref, w_ref, out_ref):
    out_ref[:] = jnp.dot(enc_ref[:].astype(jnp.float32), w_ref[:],
                         preferred_element_type=jnp.float32)

  return pl.pallas_call(
      body,
      out_shape=jax.ShapeDtypeStruct((B, cp), jnp.float32),
  )(enc, wt_pad)


@jax.jit
def kernel(x, id_weight, level_weight, classify_weight):
  id_bf = id_weight.astype(jnp.bfloat16)
  lvl_bf = level_weight.astype(jnp.bfloat16)
  xt_pad = jnp.zeros((SP, B_SC), jnp.float32).at[:S].set(x[:B_SC].T)
  lvl_pad = jnp.zeros((LP, D), jnp.bfloat16).at[:NUM_LEVELS].set(lvl_bf)
  enc_sc = _sc_encode(xt_pad, id_bf, lvl_bf)
  enc_tc = _tc_encode(x[B_SC:], id_bf, lvl_pad)
  enc = jnp.concatenate([enc_sc, enc_tc], axis=0)
  cp = 128
  wt_pad = jnp.zeros((D, cp), jnp.float32).at[:, :C].set(classify_weight.T)
  logit = _tc_classify(enc, wt_pad)
  return logit[:, :C]
